# baseline (speedup 1.0000x reference)
import functools

import jax
import jax.numpy as jnp
from jax import lax
from jax.experimental import pallas as pl
from jax.experimental.pallas import tpu as pltpu
from jax.experimental.pallas import tpu_sc as plsc

V = 1000000
B = 16384
D = 64
NC = 2
NS = 16
NW = NC * NS
BPW = B // NW
GRP = 16
NGRP = BPW // GRP
SUB = 4
NSW = GRP // SUB
SBG = 3
NSUP = 10
LANES = 16
VPE = D // LANES


def _sc_embed_lookup(X, tab_t, shared_flat):
    mesh = plsc.VectorSubcoreMesh(core_axis_name="c", subcore_axis_name="s")

    @functools.partial(
        pl.kernel,
        mesh=mesh,
        out_type=jax.ShapeDtypeStruct((B // 2, 2 * D), jnp.float32),
        compiler_params=pltpu.CompilerParams(
            use_tc_tiling_on_sc=True, needs_layout_passes=False
        ),
        scratch_types=[
            pltpu.VMEM((BPW,), jnp.int32),
            [pltpu.VMEM((SUB, D, 2 * D), jnp.float32) for _ in range(3)],
            pltpu.VMEM((SBG * 8, 2 * D), jnp.float32),
            pltpu.VMEM((D,), jnp.float32),
            pltpu.SemaphoreType.DMA,
        ],
    )
    def body(x_hbm, tab_hbm, sh_hbm, out_hbm, xr, stg, ob, sh_v, gsem):
        wid = lax.axis_index("s") * NC + lax.axis_index("c")
        base = wid * BPW
        obase = wid * (BPW // 2)

        pltpu.sync_copy(sh_hbm, sh_v)
        for j in range(4):
            pltpu.sync_copy(
                x_hbm.at[pl.ds(base + j * 128, 128)], xr.at[pl.ds(j * 128, 128)]
            )

        svs = [sh_v[pl.ds(k * LANES, LANES)] for k in range(VPE)]
        iot = lax.iota(jnp.int32, LANES)

        def fire(g, lb, buf):
            xv = xr[pl.ds(g * GRP, GRP)]
            for l in range(SUB):
                x = xv[lb + l]
                col = pl.multiple_of((x >> 7) << 7, 2 * D)
                pltpu.async_copy(
                    tab_hbm.at[pl.ds(0, D), pl.ds(col, 2 * D)],
                    stg[buf].at[l],
                    gsem,
                )

        def drain(buf):
            for l in range(SUB):
                pltpu.make_async_copy(
                    tab_hbm.at[pl.ds(0, D), pl.ds(0, 2 * D)],
                    stg[buf].at[l],
                    gsem,
                ).wait()

        def extract(g, lb, buf, row0):
            xv = xr[pl.ds(g * GRP, GRP)]
            for l in range(SUB):
                x = xv[lb + l]
                cvec = jnp.full((LANES,), x & 127, jnp.int32)
                eh = lb + l
                row = row0 + eh // 2
                lane0 = (eh % 2) * D
                for k in range(VPE):
                    v = plsc.load_gather(
                        stg[buf].at[l], [k * LANES + iot, cvec]
                    )
                    ob[row, pl.ds(lane0 + k * LANES, LANES)] = v + svs[k]

        fire(jnp.int32(0), 0, 0)
        fire(jnp.int32(0), SUB, 1)

        def superblock(it, carry):
            g0 = it * SBG
            for j in range(SBG * NSW):
                jf = j + 2
                fire(g0 + jf // NSW, (jf % NSW) * SUB, jf % 3)
                drain(j % 3)
                extract(g0 + j // NSW, (j % NSW) * SUB, j % 3, (j // NSW) * 8)
            pltpu.sync_copy(
                ob, out_hbm.at[pl.ds(obase + it * (SBG * 8), SBG * 8)]
            )
            return carry

        lax.fori_loop(0, NSUP, superblock, 0)

        ge0 = NSUP * SBG
        for j in range(2 * NSW):
            if j + 2 < 2 * NSW:
                jf = j + 2
                fire(
                    jnp.int32(ge0 + jf // NSW), (jf % NSW) * SUB, (120 + jf) % 3
                )
            drain((120 + j) % 3)
            extract(
                jnp.int32(ge0 + j // NSW), (j % NSW) * SUB, (120 + j) % 3,
                (j // NSW) * 8,
            )
        pltpu.sync_copy(
            ob.at[pl.ds(0, 16)],
            out_hbm.at[pl.ds(obase + NSUP * (SBG * 8), 16)],
        )

    return body(X, tab_t, shared_flat)


def kernel(X, embed_table, shared_embed):
    out_p = _sc_embed_lookup(X, embed_table.T, shared_embed.reshape(D))
    return out_p.reshape(B, D)


# speedup vs baseline: 1.0321x; 1.0321x over previous
import functools

import jax
import jax.numpy as jnp
from jax import lax
from jax.experimental import pallas as pl
from jax.experimental.pallas import tpu as pltpu
from jax.experimental.pallas import tpu_sc as plsc

V = 1000000
B = 16384
D = 64
NC = 2
NS = 16
NW = NC * NS
BPW = B // NW
GRP = 16
NGRP = BPW // GRP
SUB = 4
LANES = 16
VPE = D // LANES


def _sc_embed_lookup(X, tab_t, shared_flat):
    mesh = plsc.VectorSubcoreMesh(core_axis_name="c", subcore_axis_name="s")

    @functools.partial(
        pl.kernel,
        mesh=mesh,
        out_type=jax.ShapeDtypeStruct((B // 2, 2 * D), jnp.float32),
        compiler_params=pltpu.CompilerParams(
            use_tc_tiling_on_sc=True, needs_layout_passes=False
        ),
        scratch_types=[
            pltpu.VMEM((BPW,), jnp.int32),
            [pltpu.VMEM((SUB, D, 2 * D), jnp.float32) for _ in range(2)],
            pltpu.VMEM((BPW // 2, 2 * D), jnp.float32),
            pltpu.VMEM((D,), jnp.float32),
            pltpu.SemaphoreType.DMA,
        ],
    )
    def body(x_hbm, tab_hbm, sh_hbm, out_hbm, xr, stg, ob, sh_v, gsem):
        wid = lax.axis_index("s") * NC + lax.axis_index("c")
        base = wid * BPW

        pltpu.sync_copy(sh_hbm, sh_v)
        for j in range(4):
            pltpu.sync_copy(
                x_hbm.at[pl.ds(base + j * 128, 128)], xr.at[pl.ds(j * 128, 128)]
            )

        svs = [sh_v[pl.ds(k * LANES, LANES)] for k in range(VPE)]
        iot = lax.iota(jnp.int32, LANES)

        def fire(xv, lb, buf):
            for l in range(SUB):
                x = xv[lb + l]
                col = pl.multiple_of((x >> 7) << 7, 2 * D)
                pltpu.async_copy(
                    tab_hbm.at[pl.ds(0, D), pl.ds(col, 2 * D)],
                    stg[buf].at[l],
                    gsem,
                )

        def drain(buf):
            for l in range(SUB):
                pltpu.make_async_copy(
                    tab_hbm.at[pl.ds(0, D), pl.ds(0, 2 * D)],
                    stg[buf].at[l],
                    gsem,
                ).wait()

        def extract(xv, lb, buf, g):
            for l in range(SUB):
                x = xv[lb + l]
                cvec = jnp.full((LANES,), x & 127, jnp.int32)
                eh = lb + l
                row = g * (GRP // 2) + eh // 2
                lane0 = (eh % 2) * D
                for k in range(VPE):
                    v = plsc.load_gather(
                        stg[buf].at[l], [k * LANES + iot, cvec]
                    )
                    ob[row, pl.ds(lane0 + k * LANES, LANES)] = v + svs[k]

        NSW = GRP // SUB

        def group(g, carry):
            xv = xr[pl.ds(g * GRP, GRP)]
            for s in range(NSW - 1):
                fire(xv, (s + 1) * SUB, (s + 1) & 1)
                drain(s & 1)
                extract(xv, s * SUB, s & 1, g)
            gn = jnp.minimum(g + 1, NGRP - 1)
            xvn = xr[pl.ds(gn * GRP, GRP)]
            fire(xvn, 0, NSW & 1)
            drain((NSW - 1) & 1)
            extract(xv, (NSW - 1) * SUB, (NSW - 1) & 1, g)
            return carry

        xv0 = xr[pl.ds(0, GRP)]
        fire(xv0, 0, 0)
        lax.fori_loop(0, NGRP, group, 0)
        drain(NSW & 1)
        pltpu.sync_copy(ob, out_hbm.at[pl.ds(wid * (BPW // 2), BPW // 2)])

    return body(X, tab_t, shared_flat)


def kernel(X, embed_table, shared_embed):
    out_p = _sc_embed_lookup(X, embed_table.T, shared_embed.reshape(D))
    return out_p.reshape(B, D)


# speedup vs baseline: 1.0745x; 1.0411x over previous
import functools

import jax
import jax.numpy as jnp
from jax import lax
from jax.experimental import pallas as pl
from jax.experimental.pallas import tpu as pltpu
from jax.experimental.pallas import tpu_sc as plsc

V = 1000000
B = 16384
D = 64
NC = 2
NS = 16
NW = NC * NS
BPW = B // NW
GRP = 16
NGRP = BPW // GRP
GPB = 8
NBLK = NGRP // GPB
SUB = 4
NSW = GRP // SUB
LANES = 16
VPE = D // LANES


def _sc_embed_lookup(X, tab_t, shared_flat):
    mesh = plsc.VectorSubcoreMesh(core_axis_name="c", subcore_axis_name="s")

    @functools.partial(
        pl.kernel,
        mesh=mesh,
        out_type=jax.ShapeDtypeStruct((D, B), jnp.float32),
        compiler_params=pltpu.CompilerParams(
            use_tc_tiling_on_sc=True, needs_layout_passes=False
        ),
        scratch_types=[
            pltpu.VMEM((BPW,), jnp.int32),
            [pltpu.VMEM((SUB, D, 2 * D), jnp.float32) for _ in range(2)],
            [pltpu.VMEM((D, 2 * D), jnp.float32) for _ in range(2)],
            pltpu.VMEM((D,), jnp.float32),
            pltpu.SemaphoreType.DMA,
            pltpu.SemaphoreType.DMA,
        ],
    )
    def body(x_hbm, tab_hbm, sh_hbm, out_hbm, xr, stg, ob, sh_v, gsem, osem):
        wid = lax.axis_index("s") * NC + lax.axis_index("c")
        base = wid * BPW

        pltpu.sync_copy(sh_hbm, sh_v)
        for j in range(4):
            pltpu.sync_copy(
                x_hbm.at[pl.ds(base + j * 128, 128)], xr.at[pl.ds(j * 128, 128)]
            )

        svs = [sh_v[pl.ds(k * LANES, LANES)] for k in range(VPE)]
        iot = lax.iota(jnp.int32, LANES)

        def fire(g, lb, buf):
            xv = xr[pl.ds(g * GRP, GRP)]
            for l in range(SUB):
                x = xv[lb + l]
                col = pl.multiple_of((x >> 7) << 7, 2 * D)
                pltpu.async_copy(
                    tab_hbm.at[pl.ds(0, D), pl.ds(col, 2 * D)],
                    stg[buf].at[l],
                    gsem,
                )

        def drain(buf):
            for l in range(SUB):
                pltpu.make_async_copy(
                    tab_hbm.at[pl.ds(0, D), pl.ds(0, 2 * D)],
                    stg[buf].at[l],
                    gsem,
                ).wait()

        def extract(g8, g, lb, buf, obuf):
            xv = xr[pl.ds(g * GRP, GRP)]
            for l in range(SUB):
                x = xv[lb + l]
                cvec = jnp.full((LANES,), x & 127, jnp.int32)
                ovec = jnp.full((LANES,), g8 * GRP + lb + l, jnp.int32)
                for k in range(VPE):
                    v = plsc.load_gather(
                        stg[buf].at[l], [k * LANES + iot, cvec]
                    )
                    plsc.store_scatter(
                        ob[obuf], [k * LANES + iot, ovec], v + svs[k]
                    )

        def out_slice(blk):
            return out_hbm.at[pl.ds(0, D), pl.ds(base + blk * 2 * D, 2 * D)]

        fire(jnp.int32(0), 0, 0)
        for blk in range(NBLK):
            obuf = blk & 1
            if blk >= 2:
                pltpu.make_async_copy(out_slice(blk - 2), ob[obuf], osem).wait()

            def group(g8, carry):
                g = blk * GPB + g8
                for s in range(NSW - 1):
                    fire(g, (s + 1) * SUB, (s + 1) & 1)
                    drain(s & 1)
                    extract(g8, g, s * SUB, s & 1, obuf)
                gn = jnp.minimum(g + 1, NGRP - 1)
                fire(gn, 0, NSW & 1)
                drain((NSW - 1) & 1)
                extract(g8, g, (NSW - 1) * SUB, (NSW - 1) & 1, obuf)
                return carry

            lax.fori_loop(0, GPB, group, 0)
            pltpu.async_copy(ob[obuf], out_slice(blk), osem)
        drain(NSW & 1)
        pltpu.make_async_copy(out_slice(NBLK - 2), ob[NBLK & 1], osem).wait()
        pltpu.make_async_copy(out_slice(NBLK - 1), ob[(NBLK - 1) & 1], osem).wait()

    return body(X, tab_t, shared_flat)


def kernel(X, embed_table, shared_embed):
    return _sc_embed_lookup(X, embed_table.T, shared_embed.reshape(D)).T
